# Initial kernel scaffold; baseline (speedup 1.0000x reference)
import functools

import jax
import jax.numpy as jnp
from jax import lax
from jax.experimental import pallas as pl
from jax.experimental.pallas import tpu as pltpu
from jax.experimental.pallas import tpu_sc as plsc

_BATCH = 4096
_HIST = 50
_DIM = 128
_B = _BATCH * _HIST
_NC = 2
_NS = 16
_NW = _NC * _NS
_B_W = _B // _NW
_CHUNK = 128
_NCHUNK = _B_W // _CHUNK
_IDX_ROWS = _B_W // 128

_mesh = plsc.VectorSubcoreMesh(core_axis_name="c", subcore_axis_name="s")


@functools.partial(
    pl.kernel,
    mesh=_mesh,
    out_type=jax.ShapeDtypeStruct((_B, _DIM), jnp.float32),
    scratch_types=[
        pltpu.VMEM((_IDX_ROWS, 128), jnp.int32),
        pltpu.VMEM((_CHUNK, _DIM), jnp.float32),
        pltpu.SemaphoreType.DMA,
    ],
)
def _gather(x_hbm, table_hbm, out_hbm, idx_v, rows_v, sem):
    wid = lax.axis_index("s") * _NC + lax.axis_index("c")
    pltpu.sync_copy(x_hbm.at[pl.ds(wid * _IDX_ROWS, _IDX_ROWS)], idx_v)

    def step(j, carry):
        pltpu.async_copy(table_hbm.at[idx_v.at[j]], rows_v, sem).wait()
        pltpu.sync_copy(rows_v, out_hbm.at[pl.ds(wid * _B_W + j * _CHUNK, _CHUNK)])
        return carry

    lax.fori_loop(0, _NCHUNK, step, 0)


def kernel(x, table):
    x2 = x.reshape(_B // 128, 128).astype(jnp.int32)
    out = _gather(x2, table)
    return out.reshape(_BATCH, _HIST, _DIM)


# speedup vs baseline: 2.9610x; 2.9610x over previous
import functools

import jax
import jax.numpy as jnp
from jax import lax
from jax.experimental import pallas as pl
from jax.experimental.pallas import tpu as pltpu
from jax.experimental.pallas import tpu_sc as plsc

_BATCH = 4096
_HIST = 50
_DIM = 128
_B = _BATCH * _HIST
_NC = 2
_NS = 16
_NW = _NC * _NS
_B_W = _B // _NW
_CHUNK = 128
_NCHUNK = _B_W // _CHUNK

_mesh = plsc.VectorSubcoreMesh(core_axis_name="c", subcore_axis_name="s")


@functools.partial(
    pl.kernel,
    mesh=_mesh,
    out_type=jax.ShapeDtypeStruct((_B, _DIM), jnp.float32),
    scratch_types=[
        pltpu.VMEM((_B_W,), jnp.int32),
        pltpu.VMEM((_CHUNK, _DIM), jnp.float32),
        pltpu.SemaphoreType.DMA,
    ],
)
def _gather(x_hbm, table_hbm, out_hbm, idx_v, rows_v, sem):
    wid = lax.axis_index("s") * _NC + lax.axis_index("c")
    pltpu.sync_copy(x_hbm.at[pl.ds(wid * _B_W, _B_W)], idx_v)

    def step(j, carry):
        idx = idx_v.at[pl.ds(j * _CHUNK, _CHUNK)]
        pltpu.async_copy(table_hbm.at[idx], rows_v, sem).wait()
        pltpu.sync_copy(rows_v, out_hbm.at[pl.ds(wid * _B_W + j * _CHUNK, _CHUNK)])
        return carry

    lax.fori_loop(0, _NCHUNK, step, 0)


def kernel(x, table):
    x1 = x.reshape(_B).astype(jnp.int32)
    out = _gather(x1, table)
    return out.reshape(_BATCH, _HIST, _DIM)


# speedup vs baseline: 3.2944x; 1.1126x over previous
import functools

import jax
import jax.numpy as jnp
from jax import lax
from jax.experimental import pallas as pl
from jax.experimental.pallas import tpu as pltpu
from jax.experimental.pallas import tpu_sc as plsc

_BATCH = 4096
_HIST = 50
_DIM = 128
_B = _BATCH * _HIST
_NC = 2
_NS = 16
_NW = _NC * _NS
_B_W = _B // _NW
_CHUNK = 128
_NCHUNK = _B_W // _CHUNK
_NBUF = 5

_mesh = plsc.VectorSubcoreMesh(core_axis_name="c", subcore_axis_name="s")


@functools.partial(
    pl.kernel,
    mesh=_mesh,
    out_type=jax.ShapeDtypeStruct((_B, _DIM), jnp.float32),
    scratch_types=[
        pltpu.VMEM((_B_W,), jnp.int32),
    ]
    + [pltpu.VMEM((_CHUNK, _DIM), jnp.float32) for _ in range(_NBUF)]
    + [pltpu.SemaphoreType.DMA for _ in range(2 * _NBUF)],
)
def _gather(x_hbm, table_hbm, out_hbm, idx_v, *bufs_and_sems):
    rows = bufs_and_sems[:_NBUF]
    gsem = bufs_and_sems[_NBUF:2 * _NBUF]
    ssem = bufs_and_sems[2 * _NBUF:]
    wid = lax.axis_index("s") * _NC + lax.axis_index("c")
    base = wid * _B_W
    pltpu.sync_copy(x_hbm.at[pl.ds(base, _B_W)], idx_v)

    def start_gather(j, b):
        idx = idx_v.at[pl.ds(j * _CHUNK, _CHUNK)]
        pltpu.make_async_copy(table_hbm.at[idx], rows[b], gsem[b]).start()

    def wait_gather(b):
        idx = idx_v.at[pl.ds(0, _CHUNK)]
        pltpu.make_async_copy(table_hbm.at[idx], rows[b], gsem[b]).wait()

    def start_store(j, b):
        dst = out_hbm.at[pl.ds(base + j * _CHUNK, _CHUNK)]
        pltpu.make_async_copy(rows[b], dst, ssem[b]).start()

    def wait_store(b):
        dst = out_hbm.at[pl.ds(base, _CHUNK)]
        pltpu.make_async_copy(rows[b], dst, ssem[b]).wait()

    for b in range(_NBUF):
        start_gather(b, b)

    def outer(t, carry):
        for b in range(_NBUF):
            wait_gather(b)
            start_store(t * _NBUF + b, b)

        @pl.when(t < _NCHUNK // _NBUF - 1)
        def _():
            for b in range(_NBUF):
                wait_store(b)
                start_gather((t + 1) * _NBUF + b, b)

        return carry

    lax.fori_loop(0, _NCHUNK // _NBUF, outer, 0)

    for b in range(_NBUF):
        wait_store(b)


def kernel(x, table):
    x1 = x.reshape(_B).astype(jnp.int32)
    out = _gather(x1, table)
    return out.reshape(_BATCH, _HIST, _DIM)
